# baseline (speedup 1.0000x reference)
import functools

import jax
import jax.numpy as jnp
from jax import lax
from jax.experimental import pallas as pl
from jax.experimental.pallas import tpu as pltpu
from jax.experimental.pallas import tpu_sc as plsc

N_TOK = 256
N_NODE = 10000
N_TOT = N_TOK + N_NODE
E = 32768
D = 64
D_TOK = 768
N_REL = 40

NC = 2
NS = 16
NW = NC * NS

f32 = jnp.float32


def _sc_mesh():
    return plsc.VectorSubcoreMesh(
        core_axis_name="c", subcore_axis_name="s", num_cores=NC, num_subcores=NS
    )


CH = 128


@functools.lru_cache(maxsize=None)
def _make_gather(V, B, n_idx=1):
    b_per_w = B // NW
    n_ch = b_per_w // CH
    assert b_per_w % CH == 0

    def body(table_hbm, *rest):
        idx_hbms, out_hbms = rest[:n_idx], rest[n_idx : 2 * n_idx]
        idx_v, rows_v, sem = rest[2 * n_idx :]
        wid = lax.axis_index("s") * NC + lax.axis_index("c")
        base = wid * b_per_w
        for k in range(n_idx):
            for j in range(n_ch):
                pltpu.sync_copy(idx_hbms[k].at[pl.ds(base + j * CH, CH)], idx_v.at[j])
            descs = [
                pltpu.async_copy(
                    table_hbm.at[idx_v.at[j]], rows_v.at[pl.ds(j * CH, CH)], sem
                )
                for j in range(n_ch)
            ]
            for dsc in descs:
                dsc.wait()
            pltpu.sync_copy(rows_v, out_hbms[k].at[pl.ds(base, b_per_w)])

    out_t = [jax.ShapeDtypeStruct((B, D), f32) for _ in range(n_idx)]
    return pl.kernel(
        body,
        out_type=out_t[0] if n_idx == 1 else tuple(out_t),
        mesh=_sc_mesh(),
        compiler_params=pltpu.CompilerParams(use_tc_tiling_on_sc=False),
        scratch_types=[
            pltpu.VMEM((n_ch, CH), jnp.int32),
            pltpu.VMEM((b_per_w, D), f32),
            pltpu.SemaphoreType.DMA,
        ],
    )


NT_PAD = 10496
R_PER_T = NT_PAD // NS


@functools.lru_cache(maxsize=None)
def _make_scatter():
    e_per_w = E // NW
    n_ch = e_per_w // CH

    def body(msg_hbm, dst_hbm, zero_hbm, out_hbm, idx_v, rows_v, accum, sem):
        cid = lax.axis_index("c")
        sid = lax.axis_index("s")
        wid = sid * NC + cid
        pltpu.sync_copy(
            zero_hbm.at[pl.ds(sid * R_PER_T, R_PER_T)],
            accum.at[pl.ds(sid * R_PER_T, R_PER_T)],
        )
        plsc.subcore_barrier()
        for j in range(n_ch):
            pltpu.sync_copy(dst_hbm.at[pl.ds(wid * e_per_w + j * CH, CH)], idx_v.at[j])
        pltpu.async_copy(msg_hbm.at[pl.ds(wid * e_per_w, e_per_w)], rows_v, sem).wait()
        for j in range(n_ch):
            pltpu.sync_copy(
                rows_v.at[pl.ds(j * CH, CH)], accum.at[idx_v.at[j]], add=True
            )
        plsc.subcore_barrier()
        pltpu.sync_copy(
            accum.at[pl.ds(sid * R_PER_T, R_PER_T)],
            out_hbm.at[cid].at[pl.ds(sid * R_PER_T, R_PER_T)],
        )

    return pl.kernel(
        body,
        out_type=jax.ShapeDtypeStruct((NC, NT_PAD, D), f32),
        mesh=_sc_mesh(),
        compiler_params=pltpu.CompilerParams(use_tc_tiling_on_sc=False),
        scratch_types=[
            pltpu.VMEM((n_ch, CH), jnp.int32),
            pltpu.VMEM((e_per_w, D), f32),
            pltpu.VMEM_SHARED((NT_PAD, D), f32),
            pltpu.SemaphoreType.DMA,
        ],
    )


def _te_body(tok_ref, w1_ref, b1_ref, o_ref):
    o_ref[...] = (
        jnp.dot(tok_ref[...], w1_ref[...], preferred_element_type=f32) + b1_ref[...]
    )


def _te_prep(tok, W1, b1):
    return pl.pallas_call(
        _te_body,
        out_shape=jax.ShapeDtypeStruct((N_TOK, D), f32),
    )(tok, W1, b1)


BE = 512


def _msg_body(ea_ref, xj_ref, ee_ref, wen_ref, ben_ref, o_ref):
    emb = jnp.dot(ea_ref[...], ee_ref[...], preferred_element_type=f32)
    w = jnp.maximum(
        jnp.dot(emb.astype(jnp.bfloat16), wen_ref[...], preferred_element_type=f32)
        + ben_ref[...],
        0.0,
    )
    xj = xj_ref[...]
    acc = jnp.zeros((BE, D), f32)
    for d in range(D):
        acc = acc + xj[:, d : d + 1] * w[:, d * D : (d + 1) * D]
    o_ref[...] = acc


def _msg(edge_attr, x_j, edge_emb, W_en, b_en):
    return pl.pallas_call(
        _msg_body,
        grid=(E // BE,),
        in_specs=[
            pl.BlockSpec((BE, N_REL), lambda i: (i, 0)),
            pl.BlockSpec((BE, D), lambda i: (i, 0)),
            pl.BlockSpec((N_REL, D), lambda i: (0, 0)),
            pl.BlockSpec((D, D * D), lambda i: (0, 0)),
            pl.BlockSpec((D * D,), lambda i: (0,)),
        ],
        out_specs=pl.BlockSpec((BE, D), lambda i: (i, 0)),
        out_shape=jax.ShapeDtypeStruct((E, D), f32),
    )(edge_attr, x_j, edge_emb, W_en, b_en)


def _combine_body(p0_ref, p1_ref, x_ref, root_ref, bias_ref, o_ref, *, do_relu):
    v = (
        p0_ref[...]
        + p1_ref[...]
        + jnp.dot(x_ref[...], root_ref[...], preferred_element_type=f32)
        + bias_ref[...]
    )
    o_ref[...] = jnp.maximum(v, 0.0) if do_relu else v


def _combine(p0, p1, x, root, bias, do_relu):
    return pl.pallas_call(
        functools.partial(_combine_body, do_relu=do_relu),
        out_shape=jax.ShapeDtypeStruct((N_TOT, D), f32),
    )(p0, p1, x, root, bias)


BN = 5128


def _epiA_body(fx_ref, w2_ref, b2_ref, wnt_ref, bnt_ref, lab_ref, o_ref, nt_ref):
    i = pl.program_id(0)
    fx = fx_ref[...]
    o_ref[...] = jnp.dot(fx, w2_ref[...], preferred_element_type=f32) + b2_ref[...]
    logits = jnp.dot(fx, wnt_ref[...], preferred_element_type=f32) + bnt_ref[...]
    m = jnp.max(logits, axis=1, keepdims=True)
    lse = m + jnp.log(jnp.sum(jnp.exp(logits - m), axis=1, keepdims=True))
    logp = logits - lse
    oh = (lab_ref[...] == lax.broadcasted_iota(jnp.int32, (1, 3), 1)).astype(f32)
    picked = jnp.sum(logp * oh, keepdims=True)

    @pl.when(i == 0)
    def _():
        nt_ref[...] = jnp.zeros((1, 1), f32)

    nt_ref[...] += -picked / N_TOT


def _epiA(fx, W2, b2, W_nt, b_nt, labels2d):
    return pl.pallas_call(
        _epiA_body,
        grid=(N_TOT // BN,),
        in_specs=[
            pl.BlockSpec((BN, D), lambda i: (i, 0)),
            pl.BlockSpec((D, D_TOK), lambda i: (0, 0)),
            pl.BlockSpec((D_TOK,), lambda i: (0,)),
            pl.BlockSpec((D, 3), lambda i: (0, 0)),
            pl.BlockSpec((3,), lambda i: (0,)),
            pl.BlockSpec((BN, 1), lambda i: (i, 0)),
        ],
        out_specs=[
            pl.BlockSpec((BN, D_TOK), lambda i: (i, 0)),
            pl.BlockSpec((1, 1), lambda i: (0, 0)),
        ],
        out_shape=[
            jax.ShapeDtypeStruct((N_TOT, D_TOK), f32),
            jax.ShapeDtypeStruct((1, 1), f32),
        ],
    )(fx, W2, b2, W_nt, b_nt, labels2d)


BEL = 2048


def _epiB_body(ea_ref, fs_ref, fd_ref, ee_ref, kge_ref):
    i = pl.program_id(0)
    ea = ea_ref[...]
    eemb = jnp.dot(ea, ee_ref[...], preferred_element_type=f32)
    dlt = fs_ref[...] + eemb - fd_ref[...]
    mask = (jnp.sum(ea[:, N_REL - 3 :], axis=1) == 0.0).astype(f32)[:, None]
    s = jnp.sum(dlt * dlt * mask, keepdims=True)

    @pl.when(i == 0)
    def _():
        kge_ref[...] = jnp.zeros((1, 1), f32)

    kge_ref[...] += s / (E * D)


def _epiB(edge_attr, fs, fd, edge_emb):
    return pl.pallas_call(
        _epiB_body,
        grid=(E // BEL,),
        in_specs=[
            pl.BlockSpec((BEL, N_REL), lambda i: (i, 0)),
            pl.BlockSpec((BEL, D), lambda i: (i, 0)),
            pl.BlockSpec((BEL, D), lambda i: (i, 0)),
            pl.BlockSpec((N_REL, D), lambda i: (0, 0)),
        ],
        out_specs=pl.BlockSpec((1, 1), lambda i: (0, 0)),
        out_shape=jax.ShapeDtypeStruct((1, 1), f32),
    )(edge_attr, fs, fd, edge_emb)


def kernel(node_ids, edge_index, edge_attr, token_embeddings, node_type_labels,
           num_recognized_tokens, mask_out_rate,
           kg_emb, edge_emb, W_en, b_en, W1, b1, W2, b2,
           root1, bias1, root2, bias2, W_nt, b_nt):
    node_ids = node_ids.astype(jnp.int32)
    src = edge_index[0].astype(jnp.int32)
    dst = edge_index[1].astype(jnp.int32)

    B_NE = 12288
    nid_pad = jnp.concatenate([node_ids, jnp.zeros((B_NE - N_NODE,), jnp.int32)])
    ne = _make_gather(100000, B_NE)(kg_emb, nid_pad)[:N_NODE]
    te = _te_prep(token_embeddings, W1, b1)
    x0 = jnp.concatenate([te, ne], axis=0)
    wen_bf = W_en.astype(jnp.bfloat16)

    zero_init = jnp.zeros((NT_PAD, D), f32)
    gather_x = _make_gather(N_TOT, E)
    scatter = _make_scatter()

    xj1 = gather_x(x0, src)
    msg1 = _msg(edge_attr, xj1, edge_emb, wen_bf, b_en)
    p1 = scatter(msg1, dst, zero_init)
    x1 = _combine(p1[0, :N_TOT], p1[1, :N_TOT], x0, root1, bias1, True)

    xj2 = gather_x(x1, src)
    msg2 = _msg(edge_attr, xj2, edge_emb, wen_bf, b_en)
    p2 = scatter(msg2, dst, zero_init)
    fx = _combine(p2[0, :N_TOT], p2[1, :N_TOT], x1, root2, bias2, False)

    fs, fd = _make_gather(N_TOT, E, n_idx=2)(fx, src, dst)
    final_outputs, nt = _epiA(
        fx, W2, b2, W_nt, b_nt, node_type_labels.astype(jnp.int32).reshape(-1, 1)
    )
    kge = _epiB(edge_attr, fs, fd, edge_emb)
    return (
        final_outputs,
        kge.reshape(()).astype(f32),
        nt.reshape(()).astype(f32),
        jnp.float32(0.0),
    )


# speedup vs baseline: 1.3909x; 1.3909x over previous
import functools

import jax
import jax.numpy as jnp
from jax import lax
from jax.experimental import pallas as pl
from jax.experimental.pallas import tpu as pltpu
from jax.experimental.pallas import tpu_sc as plsc

N_TOK = 256
N_NODE = 10000
N_TOT = N_TOK + N_NODE
E = 32768
D = 64
D_TOK = 768
N_REL = 40

NC = 2
NS = 16
NW = NC * NS

f32 = jnp.float32


def _sc_mesh():
    return plsc.VectorSubcoreMesh(
        core_axis_name="c", subcore_axis_name="s", num_cores=NC, num_subcores=NS
    )


CH = 128


@functools.lru_cache(maxsize=None)
def _make_gather(V, B, n_idx=1):
    b_per_w = B // NW
    n_ch = b_per_w // CH
    assert b_per_w % CH == 0

    def body(table_hbm, *rest):
        idx_hbms, out_hbms = rest[:n_idx], rest[n_idx : 2 * n_idx]
        idx_v, rows_v, sem = rest[2 * n_idx :]
        wid = lax.axis_index("s") * NC + lax.axis_index("c")
        base = wid * b_per_w
        for k in range(n_idx):
            for j in range(n_ch):
                pltpu.sync_copy(idx_hbms[k].at[pl.ds(base + j * CH, CH)], idx_v.at[j])
            descs = [
                pltpu.async_copy(
                    table_hbm.at[idx_v.at[j]], rows_v.at[pl.ds(j * CH, CH)], sem
                )
                for j in range(n_ch)
            ]
            for dsc in descs:
                dsc.wait()
            pltpu.sync_copy(rows_v, out_hbms[k].at[pl.ds(base, b_per_w)])

    out_t = [jax.ShapeDtypeStruct((B, D), f32) for _ in range(n_idx)]
    return pl.kernel(
        body,
        out_type=out_t[0] if n_idx == 1 else tuple(out_t),
        mesh=_sc_mesh(),
        compiler_params=pltpu.CompilerParams(use_tc_tiling_on_sc=False),
        scratch_types=[
            pltpu.VMEM((n_ch, CH), jnp.int32),
            pltpu.VMEM((b_per_w, D), f32),
            pltpu.SemaphoreType.DMA,
        ],
    )


NT_PAD = 10496
R_PER_T = NT_PAD // NS


@functools.lru_cache(maxsize=None)
def _make_scatter():
    e_per_w = E // NW
    n_ch = e_per_w // CH

    def body(msg_hbm, dst_hbm, zero_hbm, out_hbm, idx_v, rows_v, accum, sem):
        cid = lax.axis_index("c")
        sid = lax.axis_index("s")
        wid = sid * NC + cid
        pltpu.sync_copy(
            zero_hbm.at[pl.ds(sid * R_PER_T, R_PER_T)],
            accum.at[pl.ds(sid * R_PER_T, R_PER_T)],
        )
        plsc.subcore_barrier()
        for j in range(n_ch):
            pltpu.sync_copy(dst_hbm.at[pl.ds(wid * e_per_w + j * CH, CH)], idx_v.at[j])
        pltpu.async_copy(msg_hbm.at[pl.ds(wid * e_per_w, e_per_w)], rows_v, sem).wait()
        for j in range(n_ch):
            pltpu.sync_copy(
                rows_v.at[pl.ds(j * CH, CH)], accum.at[idx_v.at[j]], add=True
            )
        plsc.subcore_barrier()
        pltpu.sync_copy(
            accum.at[pl.ds(sid * R_PER_T, R_PER_T)],
            out_hbm.at[cid].at[pl.ds(sid * R_PER_T, R_PER_T)],
        )

    return pl.kernel(
        body,
        out_type=jax.ShapeDtypeStruct((NC, NT_PAD, D), f32),
        mesh=_sc_mesh(),
        compiler_params=pltpu.CompilerParams(use_tc_tiling_on_sc=False),
        scratch_types=[
            pltpu.VMEM((n_ch, CH), jnp.int32),
            pltpu.VMEM((e_per_w, D), f32),
            pltpu.VMEM_SHARED((NT_PAD, D), f32),
            pltpu.SemaphoreType.DMA,
        ],
    )


def _te_body(tok_ref, w1_ref, b1_ref, o_ref):
    o_ref[...] = (
        jnp.dot(tok_ref[...], w1_ref[...], preferred_element_type=f32) + b1_ref[...]
    )


def _te_prep(tok, W1, b1):
    return pl.pallas_call(
        _te_body,
        out_shape=jax.ShapeDtypeStruct((N_TOK, D), f32),
    )(tok, W1, b1)


BE = 1024


def _msg_body(ea_ref, xj_ref, m_ref, t_ref, fold_ref, o_ref):
    bf = jnp.bfloat16
    z = jnp.dot(
        ea_ref[...].astype(bf), m_ref[...], preferred_element_type=f32
    ).astype(bf)
    xj_rep = jnp.dot(
        xj_ref[...].astype(bf), t_ref[...], preferred_element_type=f32
    ).astype(bf)
    p = jnp.maximum(z, 0) * xj_rep
    o_ref[...] = jnp.dot(p, fold_ref[...], preferred_element_type=f32)


def _msg(edge_attr1, x_j, M_bf, T_bf, F_bf):
    return pl.pallas_call(
        _msg_body,
        grid=(E // BE,),
        in_specs=[
            pl.BlockSpec((BE, N_REL + 1), lambda i: (i, 0)),
            pl.BlockSpec((BE, D), lambda i: (i, 0)),
            pl.BlockSpec((N_REL + 1, D * D), lambda i: (0, 0)),
            pl.BlockSpec((D, D * D), lambda i: (0, 0)),
            pl.BlockSpec((D * D, D), lambda i: (0, 0)),
        ],
        out_specs=pl.BlockSpec((BE, D), lambda i: (i, 0)),
        out_shape=jax.ShapeDtypeStruct((E, D), f32),
        compiler_params=pltpu.CompilerParams(vmem_limit_bytes=128 * 1024 * 1024),
    )(edge_attr1, x_j, M_bf, T_bf, F_bf)


def _combine_body(p0_ref, p1_ref, x_ref, root_ref, bias_ref, o_ref, *, do_relu):
    v = (
        p0_ref[...]
        + p1_ref[...]
        + jnp.dot(x_ref[...], root_ref[...], preferred_element_type=f32)
        + bias_ref[...]
    )
    o_ref[...] = jnp.maximum(v, 0.0) if do_relu else v


def _combine(p0, p1, x, root, bias, do_relu):
    return pl.pallas_call(
        functools.partial(_combine_body, do_relu=do_relu),
        out_shape=jax.ShapeDtypeStruct((N_TOT, D), f32),
    )(p0, p1, x, root, bias)


BN = 5128


def _epiA_body(fx_ref, w2_ref, b2_ref, wnt_ref, bnt_ref, lab_ref, o_ref, nt_ref):
    i = pl.program_id(0)
    fx = fx_ref[...]
    o_ref[...] = jnp.dot(fx, w2_ref[...], preferred_element_type=f32) + b2_ref[...]
    logits = jnp.dot(fx, wnt_ref[...], preferred_element_type=f32) + bnt_ref[...]
    m = jnp.max(logits, axis=1, keepdims=True)
    lse = m + jnp.log(jnp.sum(jnp.exp(logits - m), axis=1, keepdims=True))
    logp = logits - lse
    oh = (lab_ref[...] == lax.broadcasted_iota(jnp.int32, (1, 3), 1)).astype(f32)
    picked = jnp.sum(logp * oh, keepdims=True)

    @pl.when(i == 0)
    def _():
        nt_ref[...] = jnp.zeros((1, 1), f32)

    nt_ref[...] += -picked / N_TOT


def _epiA(fx, W2, b2, W_nt, b_nt, labels2d):
    return pl.pallas_call(
        _epiA_body,
        grid=(N_TOT // BN,),
        in_specs=[
            pl.BlockSpec((BN, D), lambda i: (i, 0)),
            pl.BlockSpec((D, D_TOK), lambda i: (0, 0)),
            pl.BlockSpec((D_TOK,), lambda i: (0,)),
            pl.BlockSpec((D, 3), lambda i: (0, 0)),
            pl.BlockSpec((3,), lambda i: (0,)),
            pl.BlockSpec((BN, 1), lambda i: (i, 0)),
        ],
        out_specs=[
            pl.BlockSpec((BN, D_TOK), lambda i: (i, 0)),
            pl.BlockSpec((1, 1), lambda i: (0, 0)),
        ],
        out_shape=[
            jax.ShapeDtypeStruct((N_TOT, D_TOK), f32),
            jax.ShapeDtypeStruct((1, 1), f32),
        ],
    )(fx, W2, b2, W_nt, b_nt, labels2d)


BEL = 2048


def _epiB_body(ea_ref, fs_ref, fd_ref, ee_ref, kge_ref):
    i = pl.program_id(0)
    ea = ea_ref[...]
    eemb = jnp.dot(ea, ee_ref[...], preferred_element_type=f32)
    dlt = fs_ref[...] + eemb - fd_ref[...]
    mask = (jnp.sum(ea[:, N_REL - 3 :], axis=1) == 0.0).astype(f32)[:, None]
    s = jnp.sum(dlt * dlt * mask, keepdims=True)

    @pl.when(i == 0)
    def _():
        kge_ref[...] = jnp.zeros((1, 1), f32)

    kge_ref[...] += s / (E * D)


def _epiB(edge_attr, fs, fd, edge_emb):
    return pl.pallas_call(
        _epiB_body,
        grid=(E // BEL,),
        in_specs=[
            pl.BlockSpec((BEL, N_REL), lambda i: (i, 0)),
            pl.BlockSpec((BEL, D), lambda i: (i, 0)),
            pl.BlockSpec((BEL, D), lambda i: (i, 0)),
            pl.BlockSpec((N_REL, D), lambda i: (0, 0)),
        ],
        out_specs=pl.BlockSpec((1, 1), lambda i: (0, 0)),
        out_shape=jax.ShapeDtypeStruct((1, 1), f32),
    )(edge_attr, fs, fd, edge_emb)


def kernel(node_ids, edge_index, edge_attr, token_embeddings, node_type_labels,
           num_recognized_tokens, mask_out_rate,
           kg_emb, edge_emb, W_en, b_en, W1, b1, W2, b2,
           root1, bias1, root2, bias2, W_nt, b_nt):
    node_ids = node_ids.astype(jnp.int32)
    src = edge_index[0].astype(jnp.int32)
    dst = edge_index[1].astype(jnp.int32)

    B_NE = 12288
    nid_pad = jnp.concatenate([node_ids, jnp.zeros((B_NE - N_NODE,), jnp.int32)])
    ne = _make_gather(100000, B_NE)(kg_emb, nid_pad)[:N_NODE]
    te = _te_prep(token_embeddings, W1, b1)
    x0 = jnp.concatenate([te, ne], axis=0)
    wen_p = W_en.reshape(D, D, D).transpose(0, 2, 1).reshape(D, D * D)
    bp = b_en.reshape(D, D).T.reshape(1, D * D)
    M_bf = jnp.concatenate([edge_emb @ wen_p, bp], axis=0).astype(jnp.bfloat16)
    ea1 = jnp.concatenate([edge_attr, jnp.ones((E, 1), f32)], axis=1)
    lane = jnp.arange(D * D, dtype=jnp.int32)
    T_bf = (lane[None, :] % D == jnp.arange(D, dtype=jnp.int32)[:, None]).astype(
        jnp.bfloat16
    )
    F_bf = (lane[:, None] // D == jnp.arange(D, dtype=jnp.int32)[None, :]).astype(
        jnp.bfloat16
    )

    zero_init = jnp.zeros((NT_PAD, D), f32)
    gather_x = _make_gather(N_TOT, E)
    scatter = _make_scatter()

    xj1 = gather_x(x0, src)
    msg1 = _msg(ea1, xj1, M_bf, T_bf, F_bf)
    p1 = scatter(msg1, dst, zero_init)
    x1 = _combine(p1[0, :N_TOT], p1[1, :N_TOT], x0, root1, bias1, True)

    xj2 = gather_x(x1, src)
    msg2 = _msg(ea1, xj2, M_bf, T_bf, F_bf)
    p2 = scatter(msg2, dst, zero_init)
    fx = _combine(p2[0, :N_TOT], p2[1, :N_TOT], x1, root2, bias2, False)

    fs, fd = _make_gather(N_TOT, E, n_idx=2)(fx, src, dst)
    final_outputs, nt = _epiA(
        fx, W2, b2, W_nt, b_nt, node_type_labels.astype(jnp.int32).reshape(-1, 1)
    )
    kge = _epiB(edge_attr, fs, fd, edge_emb)
    return (
        final_outputs,
        kge.reshape(()).astype(f32),
        nt.reshape(()).astype(f32),
        jnp.float32(0.0),
    )


# speedup vs baseline: 1.4075x; 1.0119x over previous
import functools

import jax
import jax.numpy as jnp
from jax import lax
from jax.experimental import pallas as pl
from jax.experimental.pallas import tpu as pltpu
from jax.experimental.pallas import tpu_sc as plsc

N_TOK = 256
N_NODE = 10000
N_TOT = N_TOK + N_NODE
E = 32768
D = 64
D_TOK = 768
N_REL = 40

NC = 2
NS = 16
NW = NC * NS

f32 = jnp.float32


def _sc_mesh():
    return plsc.VectorSubcoreMesh(
        core_axis_name="c", subcore_axis_name="s", num_cores=NC, num_subcores=NS
    )


CH = 128
DW = 128


@functools.lru_cache(maxsize=None)
def _make_gather(V, B, n_idx=1):
    b_per_w = B // NW
    ST = min(b_per_w, 512)
    n_st = b_per_w // ST
    ch_per_st = ST // CH
    assert b_per_w % ST == 0 and ST % CH == 0

    def body(table_hbm, *rest):
        idx_hbms, out_hbms = rest[:n_idx], rest[n_idx : 2 * n_idx]
        idx_v, rows_v, sem = rest[2 * n_idx :]
        wid = lax.axis_index("s") * NC + lax.axis_index("c")
        base = wid * b_per_w
        for k in range(n_idx):
            for s in range(n_st):
                sbase = base + s * ST
                for j in range(ch_per_st):
                    pltpu.sync_copy(
                        idx_hbms[k].at[pl.ds(sbase + j * CH, CH)], idx_v.at[j]
                    )
                descs = [
                    pltpu.async_copy(
                        table_hbm.at[idx_v.at[j]], rows_v.at[pl.ds(j * CH, CH)], sem
                    )
                    for j in range(ch_per_st)
                ]
                for dsc in descs:
                    dsc.wait()
                pltpu.sync_copy(rows_v, out_hbms[k].at[pl.ds(sbase, ST)])

    out_t = [jax.ShapeDtypeStruct((B, DW), f32) for _ in range(n_idx)]
    return pl.kernel(
        body,
        out_type=out_t[0] if n_idx == 1 else tuple(out_t),
        mesh=_sc_mesh(),
        scratch_types=[
            pltpu.VMEM((ch_per_st, CH), jnp.int32),
            pltpu.VMEM((ST, DW), f32),
            pltpu.SemaphoreType.DMA,
        ],
    )


NT_PAD = 10496
R_PER_T = NT_PAD // NS


@functools.lru_cache(maxsize=None)
def _make_scatter():
    e_per_w = E // NW
    n_ch = e_per_w // CH

    def body(msg_hbm, dst_hbm, zero_hbm, out_hbm, idx_v, rows_v, accum, sem):
        cid = lax.axis_index("c")
        sid = lax.axis_index("s")
        wid = sid * NC + cid
        pltpu.sync_copy(
            zero_hbm.at[pl.ds(sid * R_PER_T, R_PER_T)],
            accum.at[pl.ds(sid * R_PER_T, R_PER_T)],
        )
        plsc.subcore_barrier()
        for j in range(n_ch):
            pltpu.sync_copy(dst_hbm.at[pl.ds(wid * e_per_w + j * CH, CH)], idx_v.at[j])
        pltpu.async_copy(msg_hbm.at[pl.ds(wid * e_per_w, e_per_w)], rows_v, sem).wait()
        for j in range(n_ch):
            pltpu.sync_copy(
                rows_v.at[pl.ds(j * CH, CH)], accum.at[idx_v.at[j]], add=True
            )
        plsc.subcore_barrier()
        pltpu.sync_copy(
            accum.at[pl.ds(sid * R_PER_T, R_PER_T)],
            out_hbm.at[cid].at[pl.ds(sid * R_PER_T, R_PER_T)],
        )

    return pl.kernel(
        body,
        out_type=jax.ShapeDtypeStruct((NC, NT_PAD, D), f32),
        mesh=_sc_mesh(),
        compiler_params=pltpu.CompilerParams(use_tc_tiling_on_sc=False),
        scratch_types=[
            pltpu.VMEM((n_ch, CH), jnp.int32),
            pltpu.VMEM((e_per_w, D), f32),
            pltpu.VMEM_SHARED((NT_PAD, D), f32),
            pltpu.SemaphoreType.DMA,
        ],
    )


B_NE = 12288


def _x0_body(tok_ref, w1_ref, b1_ref, ne2_ref, par_ref, o_ref):
    te = jnp.dot(tok_ref[...], w1_ref[...], preferred_element_type=f32) + b1_ref[...]
    ne2 = ne2_ref[...]
    sel = jnp.where(par_ref[...] != 0, ne2[:, D:], ne2[:, :D])
    x = jnp.concatenate([te, sel[:N_NODE]], axis=0)
    o_ref[...] = jnp.concatenate([x, jnp.zeros((N_TOT, DW - D), f32)], axis=1)


def _x0_prep(tok, W1, b1, ne2, par):
    return pl.pallas_call(
        _x0_body,
        out_shape=jax.ShapeDtypeStruct((N_TOT, DW), f32),
    )(tok, W1, b1, ne2, par)


BE = 2048


def _msg_body(ea_ref, xj_ref, m_ref, t_ref, fold_ref, o_ref):
    bf = jnp.bfloat16
    ea1 = jnp.concatenate([ea_ref[...], jnp.ones((BE, 1), f32)], axis=1)
    z = jnp.dot(ea1.astype(bf), m_ref[...], preferred_element_type=f32).astype(bf)
    xj_rep = jnp.dot(
        xj_ref[...].astype(bf), t_ref[...], preferred_element_type=f32
    ).astype(bf)
    p = jnp.maximum(z, 0) * xj_rep
    o_ref[...] = jnp.dot(p, fold_ref[...], preferred_element_type=f32)


def _msg(edge_attr, x_j, M_bf, T_bf, F_bf):
    return pl.pallas_call(
        _msg_body,
        grid=(E // BE,),
        in_specs=[
            pl.BlockSpec((BE, N_REL), lambda i: (i, 0)),
            pl.BlockSpec((BE, DW), lambda i: (i, 0)),
            pl.BlockSpec((N_REL + 1, D * D), lambda i: (0, 0)),
            pl.BlockSpec((DW, D * D), lambda i: (0, 0)),
            pl.BlockSpec((D * D, D), lambda i: (0, 0)),
        ],
        out_specs=pl.BlockSpec((BE, D), lambda i: (i, 0)),
        out_shape=jax.ShapeDtypeStruct((E, D), f32),
        compiler_params=pltpu.CompilerParams(vmem_limit_bytes=128 * 1024 * 1024),
    )(edge_attr, x_j, M_bf, T_bf, F_bf)


def _combine_body(p0_ref, p1_ref, x_ref, root_ref, bias_ref, o_ref, *, do_relu):
    v = (
        p0_ref[...]
        + p1_ref[...]
        + jnp.dot(x_ref[...][:, :D], root_ref[...], preferred_element_type=f32)
        + bias_ref[...]
    )
    v = jnp.maximum(v, 0.0) if do_relu else v
    o_ref[...] = jnp.concatenate([v, jnp.zeros((N_TOT, DW - D), f32)], axis=1)


def _combine(p0, p1, x, root, bias, do_relu):
    return pl.pallas_call(
        functools.partial(_combine_body, do_relu=do_relu),
        out_shape=jax.ShapeDtypeStruct((N_TOT, DW), f32),
    )(p0, p1, x, root, bias)


BN = 5128


def _epiA_body(fx_ref, w2_ref, b2_ref, wnt_ref, bnt_ref, lab_ref, o_ref, nt_ref):
    i = pl.program_id(0)
    fx = fx_ref[...][:, :D]
    o_ref[...] = jnp.dot(fx, w2_ref[...], preferred_element_type=f32) + b2_ref[...]
    logits = jnp.dot(fx, wnt_ref[...], preferred_element_type=f32) + bnt_ref[...]
    m = jnp.max(logits, axis=1, keepdims=True)
    lse = m + jnp.log(jnp.sum(jnp.exp(logits - m), axis=1, keepdims=True))
    logp = logits - lse
    oh = (lab_ref[...] == lax.broadcasted_iota(jnp.int32, (1, 3), 1)).astype(f32)
    picked = jnp.sum(logp * oh, keepdims=True)

    @pl.when(i == 0)
    def _():
        nt_ref[...] = jnp.zeros((1, 1), f32)

    nt_ref[...] += -picked / N_TOT


def _epiA(fx, W2, b2, W_nt, b_nt, labels2d):
    return pl.pallas_call(
        _epiA_body,
        grid=(N_TOT // BN,),
        in_specs=[
            pl.BlockSpec((BN, DW), lambda i: (i, 0)),
            pl.BlockSpec((D, D_TOK), lambda i: (0, 0)),
            pl.BlockSpec((D_TOK,), lambda i: (0,)),
            pl.BlockSpec((D, 3), lambda i: (0, 0)),
            pl.BlockSpec((3,), lambda i: (0,)),
            pl.BlockSpec((BN, 1), lambda i: (i, 0)),
        ],
        out_specs=[
            pl.BlockSpec((BN, D_TOK), lambda i: (i, 0)),
            pl.BlockSpec((1, 1), lambda i: (0, 0)),
        ],
        out_shape=[
            jax.ShapeDtypeStruct((N_TOT, D_TOK), f32),
            jax.ShapeDtypeStruct((1, 1), f32),
        ],
    )(fx, W2, b2, W_nt, b_nt, labels2d)


BEL = 2048


def _epiB_body(ea_ref, fs_ref, fd_ref, ee_ref, kge_ref):
    i = pl.program_id(0)
    ea = ea_ref[...]
    eemb = jnp.dot(ea, ee_ref[...], preferred_element_type=f32)
    dlt = fs_ref[...][:, :D] + eemb - fd_ref[...][:, :D]
    mask = (jnp.sum(ea[:, N_REL - 3 :], axis=1) == 0.0).astype(f32)[:, None]
    s = jnp.sum(dlt * dlt * mask, keepdims=True)

    @pl.when(i == 0)
    def _():
        kge_ref[...] = jnp.zeros((1, 1), f32)

    kge_ref[...] += s / (E * D)


def _epiB(edge_attr, fs, fd, edge_emb):
    return pl.pallas_call(
        _epiB_body,
        grid=(E // BEL,),
        in_specs=[
            pl.BlockSpec((BEL, N_REL), lambda i: (i, 0)),
            pl.BlockSpec((BEL, DW), lambda i: (i, 0)),
            pl.BlockSpec((BEL, DW), lambda i: (i, 0)),
            pl.BlockSpec((N_REL, D), lambda i: (0, 0)),
        ],
        out_specs=pl.BlockSpec((1, 1), lambda i: (0, 0)),
        out_shape=jax.ShapeDtypeStruct((1, 1), f32),
    )(edge_attr, fs, fd, edge_emb)


def kernel(node_ids, edge_index, edge_attr, token_embeddings, node_type_labels,
           num_recognized_tokens, mask_out_rate,
           kg_emb, edge_emb, W_en, b_en, W1, b1, W2, b2,
           root1, bias1, root2, bias2, W_nt, b_nt):
    node_ids = node_ids.astype(jnp.int32)
    src = edge_index[0].astype(jnp.int32)
    dst = edge_index[1].astype(jnp.int32)

    kg2 = kg_emb.reshape(100000 // 2, 2 * D)
    nid_pad = jnp.concatenate([node_ids, jnp.zeros((B_NE - N_NODE,), jnp.int32)])
    ne2 = _make_gather(100000 // 2, B_NE)(kg2, nid_pad >> 1)
    par = (nid_pad & 1).reshape(-1, 1)
    x0 = _x0_prep(token_embeddings, W1, b1, ne2, par)
    wen_p = W_en.reshape(D, D, D).transpose(0, 2, 1).reshape(D, D * D)
    bp = b_en.reshape(D, D).T.reshape(1, D * D)
    M_bf = jnp.concatenate([edge_emb @ wen_p, bp], axis=0).astype(jnp.bfloat16)
    lane = jnp.arange(D * D, dtype=jnp.int32)
    row_w = jnp.arange(DW, dtype=jnp.int32)[:, None]
    T_bf = ((lane[None, :] % D == row_w) & (row_w < D)).astype(jnp.bfloat16)
    F_bf = (lane[:, None] // D == jnp.arange(D, dtype=jnp.int32)[None, :]).astype(
        jnp.bfloat16
    )

    zero_init = jnp.zeros((NT_PAD, D), f32)
    gather_x = _make_gather(N_TOT, E)
    scatter = _make_scatter()

    xj1 = gather_x(x0, src)
    msg1 = _msg(edge_attr, xj1, M_bf, T_bf, F_bf)
    p1 = scatter(msg1, dst, zero_init)
    x1 = _combine(p1[0, :N_TOT], p1[1, :N_TOT], x0, root1, bias1, True)

    xj2 = gather_x(x1, src)
    msg2 = _msg(edge_attr, xj2, M_bf, T_bf, F_bf)
    p2 = scatter(msg2, dst, zero_init)
    fx = _combine(p2[0, :N_TOT], p2[1, :N_TOT], x1, root2, bias2, False)

    fs, fd = _make_gather(N_TOT, E, n_idx=2)(fx, src, dst)
    final_outputs, nt = _epiA(
        fx, W2, b2, W_nt, b_nt, node_type_labels.astype(jnp.int32).reshape(-1, 1)
    )
    kge = _epiB(edge_attr, fs, fd, edge_emb)
    return (
        final_outputs,
        kge.reshape(()).astype(f32),
        nt.reshape(()).astype(f32),
        jnp.float32(0.0),
    )


# speedup vs baseline: 1.4190x; 1.0081x over previous
import functools

import jax
import jax.numpy as jnp
from jax import lax
from jax.experimental import pallas as pl
from jax.experimental.pallas import tpu as pltpu
from jax.experimental.pallas import tpu_sc as plsc

N_TOK = 256
N_NODE = 10000
N_TOT = N_TOK + N_NODE
E = 32768
D = 64
D_TOK = 768
N_REL = 40

NC = 2
NS = 16
NW = NC * NS

f32 = jnp.float32


def _sc_mesh():
    return plsc.VectorSubcoreMesh(
        core_axis_name="c", subcore_axis_name="s", num_cores=NC, num_subcores=NS
    )


CH = 128
DW = 128


@functools.lru_cache(maxsize=None)
def _make_gather(V, B, n_idx=1):
    b_per_w = B // NW
    ST = min(b_per_w, 512)
    n_st = b_per_w // ST
    ch_per_st = ST // CH
    assert b_per_w % ST == 0 and ST % CH == 0

    def body(table_hbm, *rest):
        idx_hbms, out_hbms = rest[:n_idx], rest[n_idx : 2 * n_idx]
        idx_v, rows_v, sem = rest[2 * n_idx :]
        wid = lax.axis_index("s") * NC + lax.axis_index("c")
        base = wid * b_per_w
        for k in range(n_idx):
            for s in range(n_st):
                sbase = base + s * ST
                for j in range(ch_per_st):
                    pltpu.sync_copy(
                        idx_hbms[k].at[pl.ds(sbase + j * CH, CH)], idx_v.at[j]
                    )
                descs = [
                    pltpu.async_copy(
                        table_hbm.at[idx_v.at[j]], rows_v.at[pl.ds(j * CH, CH)], sem
                    )
                    for j in range(ch_per_st)
                ]
                for dsc in descs:
                    dsc.wait()
                pltpu.sync_copy(rows_v, out_hbms[k].at[pl.ds(sbase, ST)])

    out_t = [jax.ShapeDtypeStruct((B, DW), f32) for _ in range(n_idx)]
    return pl.kernel(
        body,
        out_type=out_t[0] if n_idx == 1 else tuple(out_t),
        mesh=_sc_mesh(),
        scratch_types=[
            pltpu.VMEM((ch_per_st, CH), jnp.int32),
            pltpu.VMEM((ST, DW), f32),
            pltpu.SemaphoreType.DMA,
        ],
    )


NT_PAD = 10496
R_PER_T = NT_PAD // NS


@functools.lru_cache(maxsize=None)
def _make_scatter():
    e_per_w = E // NW
    n_ch = e_per_w // CH

    def body(msg_hbm, dst_hbm, zero_hbm, out_hbm, idx_v, rows_v, accum, sem):
        cid = lax.axis_index("c")
        sid = lax.axis_index("s")
        wid = sid * NC + cid
        pltpu.sync_copy(
            zero_hbm.at[pl.ds(sid * R_PER_T, R_PER_T)],
            accum.at[pl.ds(sid * R_PER_T, R_PER_T)],
        )
        plsc.subcore_barrier()
        for j in range(n_ch):
            pltpu.sync_copy(dst_hbm.at[pl.ds(wid * e_per_w + j * CH, CH)], idx_v.at[j])
        pltpu.async_copy(msg_hbm.at[pl.ds(wid * e_per_w, e_per_w)], rows_v, sem).wait()
        for j in range(n_ch):
            pltpu.sync_copy(
                rows_v.at[pl.ds(j * CH, CH)], accum.at[idx_v.at[j]], add=True
            )
        plsc.subcore_barrier()
        pltpu.sync_copy(
            accum.at[pl.ds(sid * R_PER_T, R_PER_T)],
            out_hbm.at[cid].at[pl.ds(sid * R_PER_T, R_PER_T)],
        )

    return pl.kernel(
        body,
        out_type=jax.ShapeDtypeStruct((NC, NT_PAD, D), f32),
        mesh=_sc_mesh(),
        compiler_params=pltpu.CompilerParams(use_tc_tiling_on_sc=False),
        scratch_types=[
            pltpu.VMEM((n_ch, CH), jnp.int32),
            pltpu.VMEM((e_per_w, D), f32),
            pltpu.VMEM_SHARED((NT_PAD, D), f32),
            pltpu.SemaphoreType.DMA,
        ],
    )


B_NE = 12288


def _x0_body(tok_ref, w1_ref, b1_ref, ne2_ref, par_ref, o_ref):
    te = jnp.dot(tok_ref[...], w1_ref[...], preferred_element_type=f32) + b1_ref[...]
    ne2 = ne2_ref[...]
    sel = jnp.where(par_ref[...] != 0, ne2[:, D:], ne2[:, :D])
    x = jnp.concatenate([te, sel[:N_NODE]], axis=0)
    o_ref[...] = jnp.concatenate([x, jnp.zeros((N_TOT, DW - D), f32)], axis=1)


def _x0_prep(tok, W1, b1, ne2, par):
    return pl.pallas_call(
        _x0_body,
        out_shape=jax.ShapeDtypeStruct((N_TOT, DW), f32),
    )(tok, W1, b1, ne2, par)


BE = 2048


def _msg_body(ea_ref, xj_ref, m_ref, t_ref, fold_ref, o_ref):
    bf = jnp.bfloat16
    ea1 = jnp.concatenate([ea_ref[...], jnp.ones((BE, 1), f32)], axis=1)
    z = jnp.dot(ea1.astype(bf), m_ref[...], preferred_element_type=f32).astype(bf)
    xj_rep = jnp.dot(
        xj_ref[...].astype(bf), t_ref[...], preferred_element_type=f32
    ).astype(bf)
    p = jnp.maximum(z, 0) * xj_rep
    o_ref[...] = jnp.dot(p, fold_ref[...], preferred_element_type=f32)


def _msg(edge_attr, x_j, M_bf, T_bf, F_bf):
    return pl.pallas_call(
        _msg_body,
        grid=(E // BE,),
        in_specs=[
            pl.BlockSpec((BE, N_REL), lambda i: (i, 0)),
            pl.BlockSpec((BE, DW), lambda i: (i, 0)),
            pl.BlockSpec((N_REL + 1, D * D), lambda i: (0, 0)),
            pl.BlockSpec((DW, D * D), lambda i: (0, 0)),
            pl.BlockSpec((D * D, D), lambda i: (0, 0)),
        ],
        out_specs=pl.BlockSpec((BE, D), lambda i: (i, 0)),
        out_shape=jax.ShapeDtypeStruct((E, D), f32),
        compiler_params=pltpu.CompilerParams(vmem_limit_bytes=128 * 1024 * 1024),
    )(edge_attr, x_j, M_bf, T_bf, F_bf)


def _combine_body(p0_ref, p1_ref, x_ref, root_ref, bias_ref, o_ref, *, do_relu):
    v = (
        p0_ref[...]
        + p1_ref[...]
        + jnp.dot(x_ref[...][:, :D], root_ref[...], preferred_element_type=f32)
        + bias_ref[...]
    )
    v = jnp.maximum(v, 0.0) if do_relu else v
    o_ref[...] = jnp.concatenate([v, jnp.zeros((N_TOT, DW - D), f32)], axis=1)


def _combine(p0, p1, x, root, bias, do_relu):
    return pl.pallas_call(
        functools.partial(_combine_body, do_relu=do_relu),
        out_shape=jax.ShapeDtypeStruct((N_TOT, DW), f32),
    )(p0, p1, x, root, bias)


BN = 5128


def _epiA_body(fx_ref, w2_ref, b2_ref, wnt_ref, bnt_ref, lab_ref, o_ref, nt_ref):
    i = pl.program_id(0)
    fx = fx_ref[...][:, :D]
    o_ref[...] = jnp.dot(fx, w2_ref[...], preferred_element_type=f32) + b2_ref[...]
    logits = jnp.dot(fx, wnt_ref[...], preferred_element_type=f32) + bnt_ref[...]
    m = jnp.max(logits, axis=1, keepdims=True)
    lse = m + jnp.log(jnp.sum(jnp.exp(logits - m), axis=1, keepdims=True))
    logp = logits - lse
    oh = (lab_ref[...] == lax.broadcasted_iota(jnp.int32, (1, 3), 1)).astype(f32)
    picked = jnp.sum(logp * oh, keepdims=True)

    @pl.when(i == 0)
    def _():
        nt_ref[...] = jnp.zeros((1, 1), f32)

    nt_ref[...] += -picked / N_TOT


def _epiA(fx, W2, b2, W_nt, b_nt, labels2d):
    return pl.pallas_call(
        _epiA_body,
        grid=(N_TOT // BN,),
        in_specs=[
            pl.BlockSpec((BN, DW), lambda i: (i, 0)),
            pl.BlockSpec((D, D_TOK), lambda i: (0, 0)),
            pl.BlockSpec((D_TOK,), lambda i: (0,)),
            pl.BlockSpec((D, 3), lambda i: (0, 0)),
            pl.BlockSpec((3,), lambda i: (0,)),
            pl.BlockSpec((BN, 1), lambda i: (i, 0)),
        ],
        out_specs=[
            pl.BlockSpec((BN, D_TOK), lambda i: (i, 0)),
            pl.BlockSpec((1, 1), lambda i: (0, 0)),
        ],
        out_shape=[
            jax.ShapeDtypeStruct((N_TOT, D_TOK), f32),
            jax.ShapeDtypeStruct((1, 1), f32),
        ],
    )(fx, W2, b2, W_nt, b_nt, labels2d)


BEL = 2048


def _epiB_body(ea_ref, fs_ref, fd_ref, ee_ref, kge_ref):
    i = pl.program_id(0)
    ea = ea_ref[...]
    eemb = jnp.dot(ea, ee_ref[...], preferred_element_type=f32)
    dlt = fs_ref[...][:, :D] + eemb - fd_ref[...][:, :D]
    mask = (jnp.sum(ea[:, N_REL - 3 :], axis=1) == 0.0).astype(f32)[:, None]
    s = jnp.sum(dlt * dlt * mask, keepdims=True)

    @pl.when(i == 0)
    def _():
        kge_ref[...] = jnp.zeros((1, 1), f32)

    kge_ref[...] += s / (E * D)


def _epiB(edge_attr, fs, fd, edge_emb):
    return pl.pallas_call(
        _epiB_body,
        grid=(E // BEL,),
        in_specs=[
            pl.BlockSpec((BEL, N_REL), lambda i: (i, 0)),
            pl.BlockSpec((BEL, DW), lambda i: (i, 0)),
            pl.BlockSpec((BEL, DW), lambda i: (i, 0)),
            pl.BlockSpec((N_REL, D), lambda i: (0, 0)),
        ],
        out_specs=pl.BlockSpec((1, 1), lambda i: (0, 0)),
        out_shape=jax.ShapeDtypeStruct((1, 1), f32),
    )(edge_attr, fs, fd, edge_emb)


def kernel(node_ids, edge_index, edge_attr, token_embeddings, node_type_labels,
           num_recognized_tokens, mask_out_rate,
           kg_emb, edge_emb, W_en, b_en, W1, b1, W2, b2,
           root1, bias1, root2, bias2, W_nt, b_nt):
    node_ids = node_ids.astype(jnp.int32)
    src = edge_index[0].astype(jnp.int32)
    dst = edge_index[1].astype(jnp.int32)

    kgp = jnp.pad(kg_emb, ((0, 0), (0, DW - D)))
    nid_pad = jnp.concatenate([node_ids, jnp.zeros((B_NE - N_NODE,), jnp.int32)])
    ne2 = _make_gather(100000, B_NE)(kgp, nid_pad)
    par = jnp.zeros((B_NE, 1), jnp.int32)
    x0 = _x0_prep(token_embeddings, W1, b1, ne2, par)
    wen_p = W_en.reshape(D, D, D).transpose(0, 2, 1).reshape(D, D * D)
    bp = b_en.reshape(D, D).T.reshape(1, D * D)
    M_bf = jnp.concatenate([edge_emb @ wen_p, bp], axis=0).astype(jnp.bfloat16)
    lane = jnp.arange(D * D, dtype=jnp.int32)
    row_w = jnp.arange(DW, dtype=jnp.int32)[:, None]
    T_bf = ((lane[None, :] % D == row_w) & (row_w < D)).astype(jnp.bfloat16)
    F_bf = (lane[:, None] // D == jnp.arange(D, dtype=jnp.int32)[None, :]).astype(
        jnp.bfloat16
    )

    zero_init = jnp.zeros((NT_PAD, D), f32)
    gather_x = _make_gather(N_TOT, E)
    scatter = _make_scatter()

    xj1 = gather_x(x0, src)
    msg1 = _msg(edge_attr, xj1, M_bf, T_bf, F_bf)
    p1 = scatter(msg1, dst, zero_init)
    x1 = _combine(p1[0, :N_TOT], p1[1, :N_TOT], x0, root1, bias1, True)

    xj2 = gather_x(x1, src)
    msg2 = _msg(edge_attr, xj2, M_bf, T_bf, F_bf)
    p2 = scatter(msg2, dst, zero_init)
    fx = _combine(p2[0, :N_TOT], p2[1, :N_TOT], x1, root2, bias2, False)

    fs, fd = _make_gather(N_TOT, E, n_idx=2)(fx, src, dst)
    final_outputs, nt = _epiA(
        fx, W2, b2, W_nt, b_nt, node_type_labels.astype(jnp.int32).reshape(-1, 1)
    )
    kge = _epiB(edge_attr, fs, fd, edge_emb)
    return (
        final_outputs,
        kge.reshape(()).astype(f32),
        nt.reshape(()).astype(f32),
        jnp.float32(0.0),
    )


# speedup vs baseline: 1.4237x; 1.0033x over previous
import functools

import jax
import jax.numpy as jnp
from jax import lax
from jax.experimental import pallas as pl
from jax.experimental.pallas import tpu as pltpu
from jax.experimental.pallas import tpu_sc as plsc

N_TOK = 256
N_NODE = 10000
N_TOT = N_TOK + N_NODE
E = 32768
D = 64
D_TOK = 768
N_REL = 40

NC = 2
NS = 16
NW = NC * NS

f32 = jnp.float32


def _sc_mesh():
    return plsc.VectorSubcoreMesh(
        core_axis_name="c", subcore_axis_name="s", num_cores=NC, num_subcores=NS
    )


CH = 128
DW = 128


@functools.lru_cache(maxsize=None)
def _make_gather(V, B, n_idx=1, dt=f32):
    b_per_w = B // NW
    ST = min(b_per_w, 512)
    n_st = b_per_w // ST
    ch_per_st = ST // CH
    assert b_per_w % ST == 0 and ST % CH == 0

    def body(table_hbm, *rest):
        idx_hbms, out_hbms = rest[:n_idx], rest[n_idx : 2 * n_idx]
        idx_v, rows_v, sem = rest[2 * n_idx :]
        wid = lax.axis_index("s") * NC + lax.axis_index("c")
        base = wid * b_per_w
        for k in range(n_idx):
            for s in range(n_st):
                sbase = base + s * ST
                for j in range(ch_per_st):
                    pltpu.sync_copy(
                        idx_hbms[k].at[pl.ds(sbase + j * CH, CH)], idx_v.at[j]
                    )
                descs = [
                    pltpu.async_copy(
                        table_hbm.at[idx_v.at[j]], rows_v.at[pl.ds(j * CH, CH)], sem
                    )
                    for j in range(ch_per_st)
                ]
                for dsc in descs:
                    dsc.wait()
                pltpu.sync_copy(rows_v, out_hbms[k].at[pl.ds(sbase, ST)])

    out_t = [jax.ShapeDtypeStruct((B, DW), dt) for _ in range(n_idx)]
    return pl.kernel(
        body,
        out_type=out_t[0] if n_idx == 1 else tuple(out_t),
        mesh=_sc_mesh(),
        scratch_types=[
            pltpu.VMEM((ch_per_st, CH), jnp.int32),
            pltpu.VMEM((ST, DW), dt),
            pltpu.SemaphoreType.DMA,
        ],
    )


NT_PAD = 10496
R_PER_T = NT_PAD // NS


@functools.lru_cache(maxsize=None)
def _make_scatter():
    e_per_w = E // NW
    n_ch = e_per_w // CH

    def body(msg_hbm, dst_hbm, zero_hbm, out_hbm, idx_v, rows_v, accum, sem):
        cid = lax.axis_index("c")
        sid = lax.axis_index("s")
        wid = sid * NC + cid
        pltpu.sync_copy(
            zero_hbm.at[pl.ds(sid * R_PER_T, R_PER_T)],
            accum.at[pl.ds(sid * R_PER_T, R_PER_T)],
        )
        plsc.subcore_barrier()
        for j in range(n_ch):
            pltpu.sync_copy(dst_hbm.at[pl.ds(wid * e_per_w + j * CH, CH)], idx_v.at[j])
        pltpu.async_copy(msg_hbm.at[pl.ds(wid * e_per_w, e_per_w)], rows_v, sem).wait()
        for j in range(n_ch):
            pltpu.sync_copy(
                rows_v.at[pl.ds(j * CH, CH)], accum.at[idx_v.at[j]], add=True
            )
        plsc.subcore_barrier()
        pltpu.sync_copy(
            accum.at[pl.ds(sid * R_PER_T, R_PER_T)],
            out_hbm.at[cid].at[pl.ds(sid * R_PER_T, R_PER_T)],
        )

    return pl.kernel(
        body,
        out_type=jax.ShapeDtypeStruct((NC, NT_PAD, D), f32),
        mesh=_sc_mesh(),
        compiler_params=pltpu.CompilerParams(use_tc_tiling_on_sc=False),
        scratch_types=[
            pltpu.VMEM((n_ch, CH), jnp.int32),
            pltpu.VMEM((e_per_w, D), f32),
            pltpu.VMEM_SHARED((NT_PAD, D), f32),
            pltpu.SemaphoreType.DMA,
        ],
    )


B_NE = 12288


def _x0_body(tok_ref, w1_ref, b1_ref, ne2_ref, o_ref):
    te = jnp.dot(tok_ref[...], w1_ref[...], preferred_element_type=f32) + b1_ref[...]
    ne = ne2_ref[...][:N_NODE, :D]
    x = jnp.concatenate([te, ne], axis=0)
    o_ref[...] = jnp.concatenate([x, jnp.zeros((N_TOT, DW - D), f32)], axis=1)


def _x0_prep(tok, W1, b1, ne2):
    return pl.pallas_call(
        _x0_body,
        out_shape=jax.ShapeDtypeStruct((N_TOT, DW), f32),
    )(tok, W1, b1, ne2)


BE = 2048


def _msg_body(ea_ref, xj_ref, m_ref, t_ref, fold_ref, o_ref):
    bf = jnp.bfloat16
    ea1 = jnp.concatenate([ea_ref[...], jnp.ones((BE, 1), f32)], axis=1)
    z = jnp.dot(ea1.astype(bf), m_ref[...], preferred_element_type=f32).astype(bf)
    xj_rep = jnp.dot(
        xj_ref[...].astype(bf), t_ref[...], preferred_element_type=f32
    ).astype(bf)
    p = jnp.maximum(z, 0) * xj_rep
    o_ref[...] = jnp.dot(p, fold_ref[...], preferred_element_type=f32)


def _msg(edge_attr, x_j, M_bf, T_bf, F_bf):
    return pl.pallas_call(
        _msg_body,
        grid=(E // BE,),
        in_specs=[
            pl.BlockSpec((BE, N_REL), lambda i: (i, 0)),
            pl.BlockSpec((BE, DW), lambda i: (i, 0)),
            pl.BlockSpec((N_REL + 1, D * D), lambda i: (0, 0)),
            pl.BlockSpec((DW, D * D), lambda i: (0, 0)),
            pl.BlockSpec((D * D, D), lambda i: (0, 0)),
        ],
        out_specs=pl.BlockSpec((BE, D), lambda i: (i, 0)),
        out_shape=jax.ShapeDtypeStruct((E, D), f32),
        compiler_params=pltpu.CompilerParams(vmem_limit_bytes=128 * 1024 * 1024),
    )(edge_attr, x_j, M_bf, T_bf, F_bf)


def _combine_body(p0_ref, p1_ref, x_ref, root_ref, bias_ref, o_ref, *, do_relu):
    v = (
        p0_ref[...]
        + p1_ref[...]
        + jnp.dot(x_ref[...][:, :D], root_ref[...], preferred_element_type=f32)
        + bias_ref[...]
    )
    v = jnp.maximum(v, 0.0) if do_relu else v
    o_ref[...] = jnp.concatenate([v, jnp.zeros((N_TOT, DW - D), f32)], axis=1)


def _combine(p0, p1, x, root, bias, do_relu):
    return pl.pallas_call(
        functools.partial(_combine_body, do_relu=do_relu),
        out_shape=jax.ShapeDtypeStruct((N_TOT, DW), f32),
    )(p0, p1, x, root, bias)


BN = 5128


def _epiA_body(fx_ref, w2_ref, b2_ref, wnt_ref, bnt_ref, lab_ref, o_ref, nt_ref):
    i = pl.program_id(0)
    fx = fx_ref[...][:, :D]
    o_ref[...] = jnp.dot(fx, w2_ref[...], preferred_element_type=f32) + b2_ref[...]
    logits = jnp.dot(fx, wnt_ref[...], preferred_element_type=f32) + bnt_ref[...]
    m = jnp.max(logits, axis=1, keepdims=True)
    lse = m + jnp.log(jnp.sum(jnp.exp(logits - m), axis=1, keepdims=True))
    logp = logits - lse
    oh = (lab_ref[...] == lax.broadcasted_iota(jnp.int32, (1, 3), 1)).astype(f32)
    picked = jnp.sum(logp * oh, keepdims=True)

    @pl.when(i == 0)
    def _():
        nt_ref[...] = jnp.zeros((1, 1), f32)

    nt_ref[...] += -picked / N_TOT


def _epiA(fx, W2, b2, W_nt, b_nt, labels2d):
    return pl.pallas_call(
        _epiA_body,
        grid=(N_TOT // BN,),
        in_specs=[
            pl.BlockSpec((BN, DW), lambda i: (i, 0)),
            pl.BlockSpec((D, D_TOK), lambda i: (0, 0)),
            pl.BlockSpec((D_TOK,), lambda i: (0,)),
            pl.BlockSpec((D, 3), lambda i: (0, 0)),
            pl.BlockSpec((3,), lambda i: (0,)),
            pl.BlockSpec((BN, 1), lambda i: (i, 0)),
        ],
        out_specs=[
            pl.BlockSpec((BN, D_TOK), lambda i: (i, 0)),
            pl.BlockSpec((1, 1), lambda i: (0, 0)),
        ],
        out_shape=[
            jax.ShapeDtypeStruct((N_TOT, D_TOK), f32),
            jax.ShapeDtypeStruct((1, 1), f32),
        ],
    )(fx, W2, b2, W_nt, b_nt, labels2d)


BEL = 2048


def _epiB_body(ea_ref, fs_ref, fd_ref, ee_ref, kge_ref):
    i = pl.program_id(0)
    ea = ea_ref[...]
    eemb = jnp.dot(ea, ee_ref[...], preferred_element_type=f32)
    dlt = fs_ref[...][:, :D].astype(f32) + eemb - fd_ref[...][:, :D].astype(f32)
    mask = (jnp.sum(ea[:, N_REL - 3 :], axis=1) == 0.0).astype(f32)[:, None]
    s = jnp.sum(dlt * dlt * mask, keepdims=True)

    @pl.when(i == 0)
    def _():
        kge_ref[...] = jnp.zeros((1, 1), f32)

    kge_ref[...] += s / (E * D)


def _epiB(edge_attr, fs, fd, edge_emb):
    return pl.pallas_call(
        _epiB_body,
        grid=(E // BEL,),
        in_specs=[
            pl.BlockSpec((BEL, N_REL), lambda i: (i, 0)),
            pl.BlockSpec((BEL, DW), lambda i: (i, 0)),
            pl.BlockSpec((BEL, DW), lambda i: (i, 0)),
            pl.BlockSpec((N_REL, D), lambda i: (0, 0)),
        ],
        out_specs=pl.BlockSpec((1, 1), lambda i: (0, 0)),
        out_shape=jax.ShapeDtypeStruct((1, 1), f32),
    )(edge_attr, fs, fd, edge_emb)


def kernel(node_ids, edge_index, edge_attr, token_embeddings, node_type_labels,
           num_recognized_tokens, mask_out_rate,
           kg_emb, edge_emb, W_en, b_en, W1, b1, W2, b2,
           root1, bias1, root2, bias2, W_nt, b_nt):
    node_ids = node_ids.astype(jnp.int32)
    src = edge_index[0].astype(jnp.int32)
    dst = edge_index[1].astype(jnp.int32)

    kgp = jnp.pad(kg_emb, ((0, 0), (0, DW - D)))
    nid_pad = jnp.concatenate([node_ids, jnp.zeros((B_NE - N_NODE,), jnp.int32)])
    ne2 = _make_gather(100000, B_NE)(kgp, nid_pad)
    x0 = _x0_prep(token_embeddings, W1, b1, ne2)
    wen_p = W_en.reshape(D, D, D).transpose(0, 2, 1).reshape(D, D * D)
    bp = b_en.reshape(D, D).T.reshape(1, D * D)
    M_bf = jnp.concatenate([edge_emb @ wen_p, bp], axis=0).astype(jnp.bfloat16)
    lane = jnp.arange(D * D, dtype=jnp.int32)
    row_w = jnp.arange(DW, dtype=jnp.int32)[:, None]
    T_bf = ((lane[None, :] % D == row_w) & (row_w < D)).astype(jnp.bfloat16)
    F_bf = (lane[:, None] // D == jnp.arange(D, dtype=jnp.int32)[None, :]).astype(
        jnp.bfloat16
    )

    zero_init = jnp.zeros((NT_PAD, D), f32)
    gather_x = _make_gather(N_TOT, E)
    scatter = _make_scatter()

    xj1 = gather_x(x0, src)
    msg1 = _msg(edge_attr, xj1, M_bf, T_bf, F_bf)
    p1 = scatter(msg1, dst, zero_init)
    x1 = _combine(p1[0, :N_TOT], p1[1, :N_TOT], x0, root1, bias1, True)

    xj2 = gather_x(x1, src)
    msg2 = _msg(edge_attr, xj2, M_bf, T_bf, F_bf)
    p2 = scatter(msg2, dst, zero_init)
    fx = _combine(p2[0, :N_TOT], p2[1, :N_TOT], x1, root2, bias2, False)

    fs, fd = _make_gather(N_TOT, E, n_idx=2)(fx, src, dst)
    final_outputs, nt = _epiA(
        fx, W2, b2, W_nt, b_nt, node_type_labels.astype(jnp.int32).reshape(-1, 1)
    )
    kge = _epiB(edge_attr, fs, fd, edge_emb)
    return (
        final_outputs,
        kge.reshape(()).astype(f32),
        nt.reshape(()).astype(f32),
        jnp.float32(0.0),
    )


# speedup vs baseline: 1.4355x; 1.0083x over previous
import functools

import jax
import jax.numpy as jnp
from jax import lax
from jax.experimental import pallas as pl
from jax.experimental.pallas import tpu as pltpu
from jax.experimental.pallas import tpu_sc as plsc

N_TOK = 256
N_NODE = 10000
N_TOT = N_TOK + N_NODE
E = 32768
D = 64
D_TOK = 768
N_REL = 40

NC = 2
NS = 16
NW = NC * NS

f32 = jnp.float32


def _sc_mesh():
    return plsc.VectorSubcoreMesh(
        core_axis_name="c", subcore_axis_name="s", num_cores=NC, num_subcores=NS
    )


CH = 128
DW = 128


@functools.lru_cache(maxsize=None)
def _make_gather(V, B, n_idx=1, dt=f32):
    b_per_w = B // NW
    n_ch = b_per_w // CH
    packed_idx = n_ch % 8 == 0
    assert b_per_w % CH == 0

    def body(table_hbm, *rest):
        idx_hbms, out_hbms = rest[:n_idx], rest[n_idx : 2 * n_idx]
        idx_v, rows_v, g0, g1, w0, w1 = rest[2 * n_idx :]
        gsem = (g0, g1)
        wsem = (w0, w1)
        wid = lax.axis_index("s") * NC + lax.axis_index("c")
        base = wid * b_per_w
        for k in range(n_idx):
            if packed_idx:
                pltpu.sync_copy(idx_hbms[k].at[pl.ds(wid * n_ch, n_ch)], idx_v)
            else:
                for j in range(n_ch):
                    pltpu.sync_copy(
                        idx_hbms[k].at[pl.ds(base + j * CH, CH)], idx_v.at[j]
                    )
            gds = {}
            wds = {}
            for s in range(n_ch):
                b = s & 1
                if s >= 2:
                    wds[s - 2].wait()
                gds[s] = pltpu.async_copy(
                    table_hbm.at[idx_v.at[s]], rows_v.at[b], gsem[b]
                )
                if s >= 1:
                    gds[s - 1].wait()
                    wds[s - 1] = pltpu.async_copy(
                        rows_v.at[(s - 1) & 1],
                        out_hbms[k].at[pl.ds(base + (s - 1) * CH, CH)],
                        wsem[(s - 1) & 1],
                    )
            last = n_ch - 1
            gds[last].wait()
            wds[last] = pltpu.async_copy(
                rows_v.at[last & 1],
                out_hbms[k].at[pl.ds(base + last * CH, CH)],
                wsem[last & 1],
            )
            if n_ch >= 2:
                wds[last - 1].wait()
            wds[last].wait()

    out_t = [jax.ShapeDtypeStruct((B, DW), dt) for _ in range(n_idx)]
    return pl.kernel(
        body,
        out_type=out_t[0] if n_idx == 1 else tuple(out_t),
        mesh=_sc_mesh(),
        scratch_types=[
            pltpu.VMEM((n_ch, CH), jnp.int32),
            pltpu.VMEM((2, CH, DW), dt),
            pltpu.SemaphoreType.DMA,
            pltpu.SemaphoreType.DMA,
            pltpu.SemaphoreType.DMA,
            pltpu.SemaphoreType.DMA,
        ],
    )


NT_PAD = 10496
R_PER_T = NT_PAD // NS


@functools.lru_cache(maxsize=None)
def _make_scatter():
    e_per_w = E // NW
    n_ch = e_per_w // CH

    def body(msg_hbm, dst_hbm, zero_hbm, out_hbm, idx_v, rows_v, accum, sem):
        cid = lax.axis_index("c")
        sid = lax.axis_index("s")
        wid = sid * NC + cid
        pltpu.sync_copy(
            zero_hbm.at[pl.ds(sid * R_PER_T, R_PER_T)],
            accum.at[pl.ds(sid * R_PER_T, R_PER_T)],
        )
        plsc.subcore_barrier()
        pltpu.sync_copy(dst_hbm.at[pl.ds(wid * n_ch, n_ch)], idx_v)
        pltpu.async_copy(msg_hbm.at[pl.ds(wid * e_per_w, e_per_w)], rows_v, sem).wait()
        for j in range(n_ch):
            pltpu.sync_copy(
                rows_v.at[pl.ds(j * CH, CH)], accum.at[idx_v.at[j]], add=True
            )
        plsc.subcore_barrier()
        pltpu.sync_copy(
            accum.at[pl.ds(sid * R_PER_T, R_PER_T)],
            out_hbm.at[cid].at[pl.ds(sid * R_PER_T, R_PER_T)],
        )

    return pl.kernel(
        body,
        out_type=jax.ShapeDtypeStruct((NC, NT_PAD, D), f32),
        mesh=_sc_mesh(),
        compiler_params=pltpu.CompilerParams(use_tc_tiling_on_sc=False),
        scratch_types=[
            pltpu.VMEM((n_ch, CH), jnp.int32),
            pltpu.VMEM((e_per_w, D), f32),
            pltpu.VMEM_SHARED((NT_PAD, D), f32),
            pltpu.SemaphoreType.DMA,
        ],
    )


B_NE = 12288


def _x0_body(tok_ref, w1_ref, b1_ref, ne2_ref, o_ref):
    te = jnp.dot(tok_ref[...], w1_ref[...], preferred_element_type=f32) + b1_ref[...]
    ne = ne2_ref[...][:N_NODE, :D]
    x = jnp.concatenate([te, ne], axis=0)
    o_ref[...] = jnp.concatenate([x, jnp.zeros((N_TOT, DW - D), f32)], axis=1)


def _x0_prep(tok, W1, b1, ne2):
    return pl.pallas_call(
        _x0_body,
        out_shape=jax.ShapeDtypeStruct((N_TOT, DW), f32),
    )(tok, W1, b1, ne2)


BE = 2048


def _msg_body(ea_ref, xj_ref, m_ref, t_ref, fold_ref, o_ref):
    bf = jnp.bfloat16
    ea1 = jnp.concatenate([ea_ref[...], jnp.ones((BE, 1), f32)], axis=1)
    z = jnp.dot(ea1.astype(bf), m_ref[...], preferred_element_type=f32).astype(bf)
    xj_rep = jnp.dot(
        xj_ref[...].astype(bf), t_ref[...], preferred_element_type=f32
    ).astype(bf)
    p = jnp.maximum(z, 0) * xj_rep
    o_ref[...] = jnp.dot(p, fold_ref[...], preferred_element_type=f32)


def _msg(edge_attr, x_j, M_bf, T_bf, F_bf):
    return pl.pallas_call(
        _msg_body,
        grid=(E // BE,),
        in_specs=[
            pl.BlockSpec((BE, N_REL), lambda i: (i, 0)),
            pl.BlockSpec((BE, DW), lambda i: (i, 0)),
            pl.BlockSpec((N_REL + 1, D * D), lambda i: (0, 0)),
            pl.BlockSpec((DW, D * D), lambda i: (0, 0)),
            pl.BlockSpec((D * D, D), lambda i: (0, 0)),
        ],
        out_specs=pl.BlockSpec((BE, D), lambda i: (i, 0)),
        out_shape=jax.ShapeDtypeStruct((E, D), f32),
        compiler_params=pltpu.CompilerParams(vmem_limit_bytes=128 * 1024 * 1024),
    )(edge_attr, x_j, M_bf, T_bf, F_bf)


def _combine_body(p0_ref, p1_ref, x_ref, root_ref, bias_ref, o_ref, *, do_relu):
    v = (
        p0_ref[...]
        + p1_ref[...]
        + jnp.dot(x_ref[...][:, :D], root_ref[...], preferred_element_type=f32)
        + bias_ref[...]
    )
    v = jnp.maximum(v, 0.0) if do_relu else v
    o_ref[...] = jnp.concatenate([v, jnp.zeros((N_TOT, DW - D), f32)], axis=1)


def _combine(p0, p1, x, root, bias, do_relu):
    return pl.pallas_call(
        functools.partial(_combine_body, do_relu=do_relu),
        out_shape=jax.ShapeDtypeStruct((N_TOT, DW), f32),
    )(p0, p1, x, root, bias)


BN = 5128


def _epiA_body(fx_ref, w2_ref, b2_ref, wnt_ref, bnt_ref, lab_ref, o_ref, nt_ref):
    i = pl.program_id(0)
    fx = fx_ref[...][:, :D]
    o_ref[...] = jnp.dot(fx, w2_ref[...], preferred_element_type=f32) + b2_ref[...]
    logits = jnp.dot(fx, wnt_ref[...], preferred_element_type=f32) + bnt_ref[...]
    m = jnp.max(logits, axis=1, keepdims=True)
    lse = m + jnp.log(jnp.sum(jnp.exp(logits - m), axis=1, keepdims=True))
    logp = logits - lse
    oh = (lab_ref[...] == lax.broadcasted_iota(jnp.int32, (1, 3), 1)).astype(f32)
    picked = jnp.sum(logp * oh, keepdims=True)

    @pl.when(i == 0)
    def _():
        nt_ref[...] = jnp.zeros((1, 1), f32)

    nt_ref[...] += -picked / N_TOT


def _epiA(fx, W2, b2, W_nt, b_nt, labels2d):
    return pl.pallas_call(
        _epiA_body,
        grid=(N_TOT // BN,),
        in_specs=[
            pl.BlockSpec((BN, DW), lambda i: (i, 0)),
            pl.BlockSpec((D, D_TOK), lambda i: (0, 0)),
            pl.BlockSpec((D_TOK,), lambda i: (0,)),
            pl.BlockSpec((D, 3), lambda i: (0, 0)),
            pl.BlockSpec((3,), lambda i: (0,)),
            pl.BlockSpec((BN, 1), lambda i: (i, 0)),
        ],
        out_specs=[
            pl.BlockSpec((BN, D_TOK), lambda i: (i, 0)),
            pl.BlockSpec((1, 1), lambda i: (0, 0)),
        ],
        out_shape=[
            jax.ShapeDtypeStruct((N_TOT, D_TOK), f32),
            jax.ShapeDtypeStruct((1, 1), f32),
        ],
    )(fx, W2, b2, W_nt, b_nt, labels2d)


BEL = 2048


def _epiB_body(ea_ref, fs_ref, fd_ref, ee_ref, kge_ref):
    i = pl.program_id(0)
    ea = ea_ref[...]
    eemb = jnp.dot(ea, ee_ref[...], preferred_element_type=f32)
    dlt = fs_ref[...][:, :D].astype(f32) + eemb - fd_ref[...][:, :D].astype(f32)
    mask = (jnp.sum(ea[:, N_REL - 3 :], axis=1) == 0.0).astype(f32)[:, None]
    s = jnp.sum(dlt * dlt * mask, keepdims=True)

    @pl.when(i == 0)
    def _():
        kge_ref[...] = jnp.zeros((1, 1), f32)

    kge_ref[...] += s / (E * D)


def _epiB(edge_attr, fs, fd, edge_emb):
    return pl.pallas_call(
        _epiB_body,
        grid=(E // BEL,),
        in_specs=[
            pl.BlockSpec((BEL, N_REL), lambda i: (i, 0)),
            pl.BlockSpec((BEL, DW), lambda i: (i, 0)),
            pl.BlockSpec((BEL, DW), lambda i: (i, 0)),
            pl.BlockSpec((N_REL, D), lambda i: (0, 0)),
        ],
        out_specs=pl.BlockSpec((1, 1), lambda i: (0, 0)),
        out_shape=jax.ShapeDtypeStruct((1, 1), f32),
    )(edge_attr, fs, fd, edge_emb)


def kernel(node_ids, edge_index, edge_attr, token_embeddings, node_type_labels,
           num_recognized_tokens, mask_out_rate,
           kg_emb, edge_emb, W_en, b_en, W1, b1, W2, b2,
           root1, bias1, root2, bias2, W_nt, b_nt):
    node_ids = node_ids.astype(jnp.int32)
    src = edge_index[0].astype(jnp.int32)
    dst = edge_index[1].astype(jnp.int32)

    kgp = jnp.pad(kg_emb, ((0, 0), (0, DW - D)))
    nid_pad = jnp.concatenate([node_ids, jnp.zeros((B_NE - N_NODE,), jnp.int32)])
    ne2 = _make_gather(100000, B_NE)(kgp, nid_pad)
    x0 = _x0_prep(token_embeddings, W1, b1, ne2)
    wen_p = W_en.reshape(D, D, D).transpose(0, 2, 1).reshape(D, D * D)
    bp = b_en.reshape(D, D).T.reshape(1, D * D)
    M_bf = jnp.concatenate([edge_emb @ wen_p, bp], axis=0).astype(jnp.bfloat16)
    lane = jnp.arange(D * D, dtype=jnp.int32)
    row_w = jnp.arange(DW, dtype=jnp.int32)[:, None]
    T_bf = ((lane[None, :] % D == row_w) & (row_w < D)).astype(jnp.bfloat16)
    F_bf = (lane[:, None] // D == jnp.arange(D, dtype=jnp.int32)[None, :]).astype(
        jnp.bfloat16
    )

    zero_init = jnp.zeros((NT_PAD, D), f32)
    gather_x = _make_gather(N_TOT, E)
    scatter = _make_scatter()

    src2 = src.reshape(E // CH, CH)
    dst2 = dst.reshape(E // CH, CH)
    xj1 = gather_x(x0, src2)
    msg1 = _msg(edge_attr, xj1, M_bf, T_bf, F_bf)
    p1 = scatter(msg1, dst2, zero_init)
    x1 = _combine(p1[0, :N_TOT], p1[1, :N_TOT], x0, root1, bias1, True)

    xj2 = gather_x(x1, src2)
    msg2 = _msg(edge_attr, xj2, M_bf, T_bf, F_bf)
    p2 = scatter(msg2, dst2, zero_init)
    fx = _combine(p2[0, :N_TOT], p2[1, :N_TOT], x1, root2, bias2, False)

    fs, fd = _make_gather(N_TOT, E, n_idx=2)(fx, src2, dst2)
    final_outputs, nt = _epiA(
        fx, W2, b2, W_nt, b_nt, node_type_labels.astype(jnp.int32).reshape(-1, 1)
    )
    kge = _epiB(edge_attr, fs, fd, edge_emb)
    return (
        final_outputs,
        kge.reshape(()).astype(f32),
        nt.reshape(()).astype(f32),
        jnp.float32(0.0),
    )


# speedup vs baseline: 1.4916x; 1.0391x over previous
import functools

import jax
import jax.numpy as jnp
from jax import lax
from jax.experimental import pallas as pl
from jax.experimental.pallas import tpu as pltpu
from jax.experimental.pallas import tpu_sc as plsc

N_TOK = 256
N_NODE = 10000
N_TOT = N_TOK + N_NODE
E = 32768
D = 64
D_TOK = 768
N_REL = 40

NC = 2
NS = 16
NW = NC * NS

f32 = jnp.float32


def _sc_mesh():
    return plsc.VectorSubcoreMesh(
        core_axis_name="c", subcore_axis_name="s", num_cores=NC, num_subcores=NS
    )


CH = 128
DW = 128


@functools.lru_cache(maxsize=None)
def _make_gather(V, B, n_idx=1, dt=f32, dw=DW, untiled=False):
    b_per_w = B // NW
    n_ch = b_per_w // CH
    packed_idx = n_ch % 8 == 0
    assert b_per_w % CH == 0

    def body(table_hbm, *rest):
        idx_hbms, out_hbms = rest[:n_idx], rest[n_idx : 2 * n_idx]
        idx_v, rows_v, g0, g1, w0, w1 = rest[2 * n_idx :]
        gsem = (g0, g1)
        wsem = (w0, w1)
        wid = lax.axis_index("s") * NC + lax.axis_index("c")
        base = wid * b_per_w
        for k in range(n_idx):
            if packed_idx:
                pltpu.sync_copy(idx_hbms[k].at[pl.ds(wid * n_ch, n_ch)], idx_v)
            else:
                for j in range(n_ch):
                    pltpu.sync_copy(
                        idx_hbms[k].at[pl.ds(base + j * CH, CH)], idx_v.at[j]
                    )
            gds = {}
            wds = {}
            for s in range(n_ch):
                b = s & 1
                if s >= 2:
                    wds[s - 2].wait()
                gds[s] = pltpu.async_copy(
                    table_hbm.at[idx_v.at[s]], rows_v.at[b], gsem[b]
                )
                if s >= 1:
                    gds[s - 1].wait()
                    wds[s - 1] = pltpu.async_copy(
                        rows_v.at[(s - 1) & 1],
                        out_hbms[k].at[pl.ds(base + (s - 1) * CH, CH)],
                        wsem[(s - 1) & 1],
                    )
            last = n_ch - 1
            gds[last].wait()
            wds[last] = pltpu.async_copy(
                rows_v.at[last & 1],
                out_hbms[k].at[pl.ds(base + last * CH, CH)],
                wsem[last & 1],
            )
            if n_ch >= 2:
                wds[last - 1].wait()
            wds[last].wait()

    out_t = [jax.ShapeDtypeStruct((B, dw), dt) for _ in range(n_idx)]
    cp = pltpu.CompilerParams(use_tc_tiling_on_sc=False) if untiled else None
    return pl.kernel(
        body,
        out_type=out_t[0] if n_idx == 1 else tuple(out_t),
        mesh=_sc_mesh(),
        compiler_params=cp,
        scratch_types=[
            pltpu.VMEM((n_ch, CH), jnp.int32),
            pltpu.VMEM((2, CH, dw), dt),
            pltpu.SemaphoreType.DMA,
            pltpu.SemaphoreType.DMA,
            pltpu.SemaphoreType.DMA,
            pltpu.SemaphoreType.DMA,
        ],
    )


NT_PAD = 10496
R_PER_T = NT_PAD // NS


@functools.lru_cache(maxsize=None)
def _make_scatter():
    e_per_w = E // NW
    n_ch = e_per_w // CH

    def body(msg_hbm, dst_hbm, zero_hbm, out_hbm, idx_v, rows_v, accum, sem):
        cid = lax.axis_index("c")
        sid = lax.axis_index("s")
        wid = sid * NC + cid
        pltpu.sync_copy(
            zero_hbm.at[pl.ds(sid * R_PER_T, R_PER_T)],
            accum.at[pl.ds(sid * R_PER_T, R_PER_T)],
        )
        plsc.subcore_barrier()
        pltpu.sync_copy(dst_hbm.at[pl.ds(wid * n_ch, n_ch)], idx_v)
        pltpu.async_copy(msg_hbm.at[pl.ds(wid * e_per_w, e_per_w)], rows_v, sem).wait()
        for j in range(n_ch):
            pltpu.sync_copy(
                rows_v.at[pl.ds(j * CH, CH)], accum.at[idx_v.at[j]], add=True
            )
        plsc.subcore_barrier()
        pltpu.sync_copy(
            accum.at[pl.ds(sid * R_PER_T, R_PER_T)],
            out_hbm.at[cid].at[pl.ds(sid * R_PER_T, R_PER_T)],
        )

    return pl.kernel(
        body,
        out_type=jax.ShapeDtypeStruct((NC, NT_PAD, D), f32),
        mesh=_sc_mesh(),
        compiler_params=pltpu.CompilerParams(use_tc_tiling_on_sc=False),
        scratch_types=[
            pltpu.VMEM((n_ch, CH), jnp.int32),
            pltpu.VMEM((e_per_w, D), f32),
            pltpu.VMEM_SHARED((NT_PAD, D), f32),
            pltpu.SemaphoreType.DMA,
        ],
    )


B_NE = 12288


def _x0_body(tok_ref, w1_ref, b1_ref, ne2_ref, o_ref):
    te = jnp.dot(tok_ref[...], w1_ref[...], preferred_element_type=f32) + b1_ref[...]
    ne = ne2_ref[...][:N_NODE]
    x = jnp.concatenate([te, ne], axis=0)
    o_ref[...] = jnp.concatenate([x, jnp.zeros((N_TOT, DW - D), f32)], axis=1)


def _x0_prep(tok, W1, b1, ne2):
    return pl.pallas_call(
        _x0_body,
        out_shape=jax.ShapeDtypeStruct((N_TOT, DW), f32),
    )(tok, W1, b1, ne2)


BE = 2048


def _msg_body(ea_ref, xj_ref, m_ref, t_ref, fold_ref, o_ref):
    bf = jnp.bfloat16
    ea1 = jnp.concatenate([ea_ref[...], jnp.ones((BE, 1), f32)], axis=1)
    z = jnp.dot(ea1.astype(bf), m_ref[...], preferred_element_type=f32).astype(bf)
    xj_rep = jnp.dot(
        xj_ref[...].astype(bf), t_ref[...], preferred_element_type=f32
    ).astype(bf)
    p = jnp.maximum(z, 0) * xj_rep
    o_ref[...] = jnp.dot(p, fold_ref[...], preferred_element_type=f32)


def _msg(edge_attr, x_j, M_bf, T_bf, F_bf):
    return pl.pallas_call(
        _msg_body,
        grid=(E // BE,),
        in_specs=[
            pl.BlockSpec((BE, N_REL), lambda i: (i, 0)),
            pl.BlockSpec((BE, DW), lambda i: (i, 0)),
            pl.BlockSpec((N_REL + 1, D * D), lambda i: (0, 0)),
            pl.BlockSpec((DW, D * D), lambda i: (0, 0)),
            pl.BlockSpec((D * D, D), lambda i: (0, 0)),
        ],
        out_specs=pl.BlockSpec((BE, D), lambda i: (i, 0)),
        out_shape=jax.ShapeDtypeStruct((E, D), f32),
        compiler_params=pltpu.CompilerParams(vmem_limit_bytes=128 * 1024 * 1024),
    )(edge_attr, x_j, M_bf, T_bf, F_bf)


def _combine_body(p0_ref, p1_ref, x_ref, root_ref, bias_ref, o_ref, *, do_relu):
    v = (
        p0_ref[...]
        + p1_ref[...]
        + jnp.dot(x_ref[...][:, :D], root_ref[...], preferred_element_type=f32)
        + bias_ref[...]
    )
    v = jnp.maximum(v, 0.0) if do_relu else v
    o_ref[...] = jnp.concatenate([v, jnp.zeros((N_TOT, DW - D), f32)], axis=1)


def _combine(p0, p1, x, root, bias, do_relu):
    return pl.pallas_call(
        functools.partial(_combine_body, do_relu=do_relu),
        out_shape=jax.ShapeDtypeStruct((N_TOT, DW), f32),
    )(p0, p1, x, root, bias)


BN = 5128


def _epiA_body(fx_ref, w2_ref, b2_ref, wnt_ref, bnt_ref, lab_ref, o_ref, nt_ref):
    i = pl.program_id(0)
    fx = fx_ref[...][:, :D]
    o_ref[...] = jnp.dot(fx, w2_ref[...], preferred_element_type=f32) + b2_ref[...]
    logits = jnp.dot(fx, wnt_ref[...], preferred_element_type=f32) + bnt_ref[...]
    m = jnp.max(logits, axis=1, keepdims=True)
    lse = m + jnp.log(jnp.sum(jnp.exp(logits - m), axis=1, keepdims=True))
    logp = logits - lse
    oh = (lab_ref[...] == lax.broadcasted_iota(jnp.int32, (1, 3), 1)).astype(f32)
    picked = jnp.sum(logp * oh, keepdims=True)

    @pl.when(i == 0)
    def _():
        nt_ref[...] = jnp.zeros((1, 1), f32)

    nt_ref[...] += -picked / N_TOT


def _epiA(fx, W2, b2, W_nt, b_nt, labels2d):
    return pl.pallas_call(
        _epiA_body,
        grid=(N_TOT // BN,),
        in_specs=[
            pl.BlockSpec((BN, DW), lambda i: (i, 0)),
            pl.BlockSpec((D, D_TOK), lambda i: (0, 0)),
            pl.BlockSpec((D_TOK,), lambda i: (0,)),
            pl.BlockSpec((D, 3), lambda i: (0, 0)),
            pl.BlockSpec((3,), lambda i: (0,)),
            pl.BlockSpec((BN, 1), lambda i: (i, 0)),
        ],
        out_specs=[
            pl.BlockSpec((BN, D_TOK), lambda i: (i, 0)),
            pl.BlockSpec((1, 1), lambda i: (0, 0)),
        ],
        out_shape=[
            jax.ShapeDtypeStruct((N_TOT, D_TOK), f32),
            jax.ShapeDtypeStruct((1, 1), f32),
        ],
    )(fx, W2, b2, W_nt, b_nt, labels2d)


BEL = 2048


def _epiB_body(ea_ref, fs_ref, fd_ref, ee_ref, kge_ref):
    i = pl.program_id(0)
    ea = ea_ref[...]
    eemb = jnp.dot(ea, ee_ref[...], preferred_element_type=f32)
    dlt = fs_ref[...][:, :D].astype(f32) + eemb - fd_ref[...][:, :D].astype(f32)
    mask = (jnp.sum(ea[:, N_REL - 3 :], axis=1) == 0.0).astype(f32)[:, None]
    s = jnp.sum(dlt * dlt * mask, keepdims=True)

    @pl.when(i == 0)
    def _():
        kge_ref[...] = jnp.zeros((1, 1), f32)

    kge_ref[...] += s / (E * D)


def _epiB(edge_attr, fs, fd, edge_emb):
    return pl.pallas_call(
        _epiB_body,
        grid=(E // BEL,),
        in_specs=[
            pl.BlockSpec((BEL, N_REL), lambda i: (i, 0)),
            pl.BlockSpec((BEL, DW), lambda i: (i, 0)),
            pl.BlockSpec((BEL, DW), lambda i: (i, 0)),
            pl.BlockSpec((N_REL, D), lambda i: (0, 0)),
        ],
        out_specs=pl.BlockSpec((1, 1), lambda i: (0, 0)),
        out_shape=jax.ShapeDtypeStruct((1, 1), f32),
    )(edge_attr, fs, fd, edge_emb)


def kernel(node_ids, edge_index, edge_attr, token_embeddings, node_type_labels,
           num_recognized_tokens, mask_out_rate,
           kg_emb, edge_emb, W_en, b_en, W1, b1, W2, b2,
           root1, bias1, root2, bias2, W_nt, b_nt):
    node_ids = node_ids.astype(jnp.int32)
    src = edge_index[0].astype(jnp.int32)
    dst = edge_index[1].astype(jnp.int32)

    nid_pad = jnp.concatenate([node_ids, jnp.zeros((B_NE - N_NODE,), jnp.int32)])
    ne2 = _make_gather(100000, B_NE, dw=D, untiled=True)(kg_emb, nid_pad)
    x0 = _x0_prep(token_embeddings, W1, b1, ne2)
    wen_p = W_en.reshape(D, D, D).transpose(0, 2, 1).reshape(D, D * D)
    bp = b_en.reshape(D, D).T.reshape(1, D * D)
    M_bf = jnp.concatenate([edge_emb @ wen_p, bp], axis=0).astype(jnp.bfloat16)
    lane = jnp.arange(D * D, dtype=jnp.int32)
    row_w = jnp.arange(DW, dtype=jnp.int32)[:, None]
    T_bf = ((lane[None, :] % D == row_w) & (row_w < D)).astype(jnp.bfloat16)
    F_bf = (lane[:, None] // D == jnp.arange(D, dtype=jnp.int32)[None, :]).astype(
        jnp.bfloat16
    )

    zero_init = jnp.zeros((NT_PAD, D), f32)
    gather_x = _make_gather(N_TOT, E)
    scatter = _make_scatter()

    src2 = src.reshape(E // CH, CH)
    dst2 = dst.reshape(E // CH, CH)
    xj1 = gather_x(x0, src2)
    msg1 = _msg(edge_attr, xj1, M_bf, T_bf, F_bf)
    p1 = scatter(msg1, dst2, zero_init)
    x1 = _combine(p1[0, :N_TOT], p1[1, :N_TOT], x0, root1, bias1, True)

    xj2 = gather_x(x1, src2)
    msg2 = _msg(edge_attr, xj2, M_bf, T_bf, F_bf)
    p2 = scatter(msg2, dst2, zero_init)
    fx = _combine(p2[0, :N_TOT], p2[1, :N_TOT], x1, root2, bias2, False)

    fs, fd = _make_gather(N_TOT, E, n_idx=2)(fx, src2, dst2)
    final_outputs, nt = _epiA(
        fx, W2, b2, W_nt, b_nt, node_type_labels.astype(jnp.int32).reshape(-1, 1)
    )
    kge = _epiB(edge_attr, fs, fd, edge_emb)
    return (
        final_outputs,
        kge.reshape(()).astype(f32),
        nt.reshape(()).astype(f32),
        jnp.float32(0.0),
    )
